# Initial kernel scaffold; baseline (speedup 1.0000x reference)
import jax
import jax.numpy as jnp
from jax.experimental import pallas as pl


def kernel(x, pattern_ids, content_w, pattern_w):
    raise NotImplementedError("write your pallas kernel here")


# speedup vs baseline: 1.4939x; 1.4939x over previous
import functools

import jax
import jax.numpy as jnp
from jax.experimental import pallas as pl
from jax.experimental.pallas import tpu as pltpu

DIM = 2048
NUM_EXPERTS = 64
NUM_PATTERNS = 16
TOP_K = 2
BATCH = 4
SEQ = 2048

TS = 512


def _router_kernel(pattern_ids_ref, x_ref, w_ref, pattern_w_ref,
                   logits_ref, wts_ref, idx_ref):
    b = pl.program_id(0)
    xt = x_ref[0]
    logits = jax.lax.dot_general(
        xt, w_ref[...],
        dimension_numbers=(((1,), (1,)), ((), ())),
        preferred_element_type=jnp.float32,
    )
    pid = pattern_ids_ref[b]
    bias = pattern_w_ref[pid, :]
    logits = logits + bias[None, :]
    logits_ref[0] = logits

    eids = jax.lax.broadcasted_iota(jnp.int32, (TS, NUM_EXPERTS), 1)
    m1 = jnp.max(logits, axis=1)
    i1 = jnp.argmax(logits, axis=1).astype(jnp.int32)
    masked = jnp.where(eids == i1[:, None], -jnp.inf, logits)
    m2 = jnp.max(masked, axis=1)
    i2 = jnp.argmax(masked, axis=1).astype(jnp.int32)
    e = jnp.exp(m2 - m1)
    w1 = 1.0 / (1.0 + e)
    w2 = e / (1.0 + e)
    wts_ref[0] = jnp.stack([w1, w2], axis=-1)
    idx_ref[0] = jnp.stack([i1, i2], axis=-1)


@jax.jit
def kernel(x, pattern_ids, content_w, pattern_w):
    grid = (BATCH, SEQ // TS)
    out_shapes = (
        jax.ShapeDtypeStruct((BATCH, SEQ, NUM_EXPERTS), jnp.float32),
        jax.ShapeDtypeStruct((BATCH, SEQ, TOP_K), jnp.float32),
        jax.ShapeDtypeStruct((BATCH, SEQ, TOP_K), jnp.int32),
    )
    logits, wts, idx = pl.pallas_call(
        _router_kernel,
        grid=grid,
        in_specs=[
            pl.BlockSpec(memory_space=pltpu.SMEM),
            pl.BlockSpec((1, TS, DIM), lambda b, s: (b, s, 0)),
            pl.BlockSpec((NUM_EXPERTS, DIM), lambda b, s: (0, 0)),
            pl.BlockSpec((NUM_PATTERNS, NUM_EXPERTS), lambda b, s: (0, 0)),
        ],
        out_specs=(
            pl.BlockSpec((1, TS, NUM_EXPERTS), lambda b, s: (b, s, 0)),
            pl.BlockSpec((1, TS, TOP_K), lambda b, s: (b, s, 0)),
            pl.BlockSpec((1, TS, TOP_K), lambda b, s: (b, s, 0)),
        ),
        out_shape=out_shapes,
    )(pattern_ids.astype(jnp.int32), x, content_w, pattern_w)
    return (wts, idx, logits)


# speedup vs baseline: 1.6641x; 1.1139x over previous
import functools

import jax
import jax.numpy as jnp
from jax.experimental import pallas as pl
from jax.experimental.pallas import tpu as pltpu

DIM = 2048
NUM_EXPERTS = 64
NUM_PATTERNS = 16
TOP_K = 2
BATCH = 4
SEQ = 2048

TS = 1024


def _router_kernel(pattern_ids_ref, x_ref, w_ref, pattern_w_ref,
                   logits_ref, wts_ref, idx_ref):
    b = pl.program_id(0)
    xt = x_ref[0]
    logits = jax.lax.dot_general(
        xt, w_ref[...],
        dimension_numbers=(((1,), (1,)), ((), ())),
        preferred_element_type=jnp.float32,
    )
    pid = pattern_ids_ref[b]
    bias = pattern_w_ref[pid, :]
    logits = logits + bias[None, :]
    logits_ref[0] = logits

    eids = jax.lax.broadcasted_iota(jnp.int32, (TS, NUM_EXPERTS), 1)
    m1 = jnp.max(logits, axis=1)
    i1 = jnp.argmax(logits, axis=1).astype(jnp.int32)
    masked = jnp.where(eids == i1[:, None], -jnp.inf, logits)
    m2 = jnp.max(masked, axis=1)
    i2 = jnp.argmax(masked, axis=1).astype(jnp.int32)
    e = jnp.exp(m2 - m1)
    w1 = 1.0 / (1.0 + e)
    w2 = e / (1.0 + e)
    wts_ref[0] = jnp.stack([w1, w2], axis=-1)
    idx_ref[0] = jnp.stack([i1, i2], axis=-1)


@jax.jit
def kernel(x, pattern_ids, content_w, pattern_w):
    grid = (BATCH, SEQ // TS)
    out_shapes = (
        jax.ShapeDtypeStruct((BATCH, SEQ, NUM_EXPERTS), jnp.float32),
        jax.ShapeDtypeStruct((BATCH, SEQ, TOP_K), jnp.float32),
        jax.ShapeDtypeStruct((BATCH, SEQ, TOP_K), jnp.int32),
    )
    logits, wts, idx = pl.pallas_call(
        _router_kernel,
        grid=grid,
        in_specs=[
            pl.BlockSpec(memory_space=pltpu.SMEM),
            pl.BlockSpec((1, TS, DIM), lambda b, s: (b, s, 0)),
            pl.BlockSpec((NUM_EXPERTS, DIM), lambda b, s: (0, 0)),
            pl.BlockSpec((NUM_PATTERNS, NUM_EXPERTS), lambda b, s: (0, 0)),
        ],
        out_specs=(
            pl.BlockSpec((1, TS, NUM_EXPERTS), lambda b, s: (b, s, 0)),
            pl.BlockSpec((1, TS, TOP_K), lambda b, s: (b, s, 0)),
            pl.BlockSpec((1, TS, TOP_K), lambda b, s: (b, s, 0)),
        ),
        out_shape=out_shapes,
    )(pattern_ids.astype(jnp.int32), x, content_w, pattern_w)
    return (wts, idx, logits)
